# baseline (speedup 1.0000x reference)
import functools

import jax
import jax.numpy as jnp
import numpy as np
from jax import lax
from jax.experimental import pallas as pl
from jax.experimental.pallas import tpu as pltpu
from jax.experimental.pallas import tpu_sc as plsc

_P = 1.0
_LB = 0.8
_HB = 1.2
_F = 4096
_N_TS = 65536
_D = 1024

with jax.default_device(jax.local_devices(backend="cpu")[0]):
    _key = jax.random.key(42)
    _k1, _k2, _k3 = jax.random.split(_key, 3)
    _coin = float(jax.random.uniform(_k1, ()))
    _selection = np.asarray(jax.random.choice(_k2, _N_TS, (_F,), replace=False))
    _factor = float((_HB - _LB) * jax.random.uniform(_k3, ()) + _LB)
    _apply = bool(_coin < _P)

_NC = 2
_NS = 16
_NW = _NC * _NS
_RPW = _N_TS // _NW
_CH = 32
_LANES = 16

_owner = _selection // _RPW
_groups = [_selection[_owner == w].astype(np.int32) for w in range(_NW)]
_maxk = max(len(g) for g in _groups)
_NCH = -(-_maxk // _CH)
_owned = np.empty((_NW, _NCH, _CH), np.int32)
for _w, _g in enumerate(_groups):
    _pad = np.full(_NCH * _CH, _g[0], np.int32)
    _pad[: len(_g)] = _g
    _owned[_w] = _pad.reshape(_NCH, _CH)

_mesh = plsc.VectorSubcoreMesh(core_axis_name="c", subcore_axis_name="s")


@functools.partial(
    pl.kernel,
    out_type=jax.ShapeDtypeStruct((_N_TS, _D), jnp.float32),
    mesh=_mesh,
    scratch_types=[
        pltpu.VMEM((_NCH, _CH), jnp.int32),
        pltpu.VMEM((_CH, _D), jnp.float32),
        pltpu.SemaphoreType.DMA,
        pltpu.SemaphoreType.DMA,
    ],
)
def _sc_scale(data_hbm, owned_hbm, out_hbm, idx_v, rows_v, csem, gsem):
    wid = lax.axis_index("s") * _NC + lax.axis_index("c")
    base = wid * _RPW
    slab = pl.ds(base, _RPW)
    cp = pltpu.async_copy(data_hbm.at[slab], out_hbm.at[slab], csem)
    pltpu.sync_copy(owned_hbm.at[wid], idx_v)
    cp.wait()
    for c in range(_NCH):
        idx_c = idx_v.at[c]
        pltpu.async_copy(data_hbm.at[idx_c], rows_v, gsem).wait()

        def _scale_row(r, carry):
            for j in range(_D // _LANES):
                sl = pl.ds(j * _LANES, _LANES)
                rows_v[r, sl] = rows_v[r, sl] * _factor
            return carry

        lax.fori_loop(0, _CH, _scale_row, 0)
        pltpu.async_copy(rows_v, out_hbm.at[idx_c], gsem).wait()


def kernel(data):
    if not _apply:
        return data
    owned = jnp.asarray(_owned)
    return _sc_scale(data, owned)


# speedup vs baseline: 33.3457x; 33.3457x over previous
import functools

import jax
import jax.numpy as jnp
import numpy as np
from jax import lax
from jax.experimental import pallas as pl
from jax.experimental.pallas import tpu as pltpu
from jax.experimental.pallas import tpu_sc as plsc

_P = 1.0
_LB = 0.8
_HB = 1.2
_F = 4096
_N_TS = 65536
_D = 1024

with jax.default_device(jax.local_devices(backend="cpu")[0]):
    _key = jax.random.key(42)
    _k1, _k2, _k3 = jax.random.split(_key, 3)
    _coin = float(jax.random.uniform(_k1, ()))
    _selection = np.asarray(jax.random.choice(_k2, _N_TS, (_F,), replace=False))
    _factor = float((_HB - _LB) * jax.random.uniform(_k3, ()) + _LB)
    _apply = bool(_coin < _P)

_NC = 2
_NS = 16
_NW = _NC * _NS
_RPW = _N_TS // _NW
_CH = 32
_LANES = 16

_owner = _selection // _RPW
_groups = [_selection[_owner == w].astype(np.int32) for w in range(_NW)]
_maxk = max(len(g) for g in _groups)
_NCH = -(-_maxk // _CH)
_owned = np.empty((_NW, _NCH, _CH), np.int32)
for _w, _g in enumerate(_groups):
    _pad = np.full(_NCH * _CH, _g[0], np.int32)
    _pad[: len(_g)] = _g
    _owned[_w] = _pad.reshape(_NCH, _CH)

_mesh = plsc.VectorSubcoreMesh(core_axis_name="c", subcore_axis_name="s")


_CCH = 32
_NCHK = _RPW // _CCH


@functools.partial(
    pl.kernel,
    out_type=jax.ShapeDtypeStruct((_N_TS, _D), jnp.float32),
    mesh=_mesh,
    scratch_types=[
        pltpu.VMEM((_NCH, _CH), jnp.int32),
        pltpu.VMEM((_CCH, _D), jnp.float32),
        pltpu.VMEM((_CCH, _D), jnp.float32),
        pltpu.SemaphoreType.DMA,
        pltpu.SemaphoreType.DMA,
        pltpu.SemaphoreType.DMA,
        pltpu.SemaphoreType.DMA,
        pltpu.SemaphoreType.DMA,
    ],
)
def _sc_scale(data_hbm, owned_hbm, out_hbm, idx_v, buf0, buf1, rs0, rs1,
              ws0, ws1, gsem):
    wid = lax.axis_index("s") * _NC + lax.axis_index("c")
    base = wid * _RPW
    bufs = (buf0, buf1)
    rsems = (rs0, rs1)
    wsems = (ws0, ws1)

    def _sl(c):
        return pl.ds(base + c * _CCH, _CCH)

    def _read(c, b):
        return pltpu.async_copy(data_hbm.at[_sl(c)], bufs[b], rsems[b])

    def _write(c, b):
        return pltpu.async_copy(bufs[b], out_hbm.at[_sl(c)], wsems[b])

    def _read_wait(c, b):
        pltpu.make_async_copy(data_hbm.at[_sl(c)], bufs[b], rsems[b]).wait()

    def _write_wait(c, b):
        pltpu.make_async_copy(bufs[b], out_hbm.at[_sl(c)], wsems[b]).wait()

    pltpu.sync_copy(owned_hbm.at[wid], idx_v)
    _read(0, 0)

    def _pair(g, carry):
        c0 = 2 * g
        _read_wait(c0, 0)

        @pl.when(g > 0)
        def _():
            _write_wait(c0 - 1, 1)

        _read(c0 + 1, 1)
        _write(c0, 0)
        _read_wait(c0 + 1, 1)

        @pl.when(g < _NCHK // 2 - 1)
        def _():
            _write_wait(c0, 0)
            _read(c0 + 2, 0)

        _write(c0 + 1, 1)
        return carry

    lax.fori_loop(0, _NCHK // 2, _pair, 0)
    _write_wait(_NCHK - 2, 0)
    _write_wait(_NCHK - 1, 1)

    for c in range(_NCH):
        idx_c = idx_v.at[c]
        pltpu.async_copy(data_hbm.at[idx_c], buf0, gsem).wait()

        def _scale_row(r, carry):
            for j in range(_D // _LANES):
                sl = pl.ds(j * _LANES, _LANES)
                buf0[r, sl] = buf0[r, sl] * _factor
            return carry

        lax.fori_loop(0, _CH, _scale_row, 0)
        pltpu.async_copy(buf0, out_hbm.at[idx_c], gsem).wait()


def kernel(data):
    if not _apply:
        return data
    owned = jnp.asarray(_owned)
    return _sc_scale(data, owned)


# speedup vs baseline: 38.5731x; 1.1568x over previous
import functools

import jax
import jax.numpy as jnp
import numpy as np
from jax import lax
from jax.experimental import pallas as pl
from jax.experimental.pallas import tpu as pltpu
from jax.experimental.pallas import tpu_sc as plsc
from jax._src.pallas import mpmd as _mpmd

_P = 1.0
_LB = 0.8
_HB = 1.2
_F = 4096
_N_TS = 65536
_D = 1024

with jax.default_device(jax.local_devices(backend="cpu")[0]):
    _key = jax.random.key(42)
    _k1, _k2, _k3 = jax.random.split(_key, 3)
    _coin = float(jax.random.uniform(_k1, ()))
    _selection = np.asarray(jax.random.choice(_k2, _N_TS, (_F,), replace=False))
    _factor = float((_HB - _LB) * jax.random.uniform(_k3, ()) + _LB)
    _apply = bool(_coin < _P)

_NC = 2
_NS = 16
_NW = _NC * _NS
_RPW = _N_TS // _NW
_CH = 32
_LANES = 16

_owner = _selection // _RPW
_groups = [_selection[_owner == w].astype(np.int32) for w in range(_NW)]
_maxk = max(len(g) for g in _groups)
_NCH = -(-_maxk // _CH)
_owned = np.empty((_NW, _NCH, _CH), np.int32)
for _w, _g in enumerate(_groups):
    _pad = np.full(_NCH * _CH, _g[0], np.int32)
    _pad[: len(_g)] = _g
    _owned[_w] = _pad.reshape(_NCH, _CH)

_ROWS = 2048


def _tc_copy_body(x_ref, o_ref):
    o_ref[...] = x_ref[...]


def _tc_copy(data):
    return pl.pallas_call(
        _tc_copy_body,
        grid=(_N_TS // _ROWS,),
        in_specs=[pl.BlockSpec((_ROWS, _D), lambda i: (i, 0))],
        out_specs=pl.BlockSpec((_ROWS, _D), lambda i: (i, 0)),
        out_shape=jax.ShapeDtypeStruct((_N_TS, _D), jnp.float32),
    )(data)


_mesh = plsc.VectorSubcoreMesh(core_axis_name="c", subcore_axis_name="s")


def _sc_scatter_body(copied_hbm, data_hbm, owned_hbm, out_hbm, idx_v, buf0,
                     buf1, g0, g1, s0, s1):
    del copied_hbm
    wid = lax.axis_index("s") * _NC + lax.axis_index("c")
    bufs = (buf0, buf1)
    gsems = (g0, g1)
    ssems = (s0, s1)

    def _gather(c, b):
        pltpu.async_copy(data_hbm.at[idx_v.at[c]], bufs[b], gsems[b])

    def _gather_wait(c, b):
        pltpu.make_async_copy(data_hbm.at[idx_v.at[c]], bufs[b],
                              gsems[b]).wait()

    def _scatter(c, b):
        pltpu.async_copy(bufs[b], out_hbm.at[idx_v.at[c]], ssems[b])

    def _scatter_wait(c, b):
        pltpu.make_async_copy(bufs[b], out_hbm.at[idx_v.at[c]],
                              ssems[b]).wait()

    pltpu.sync_copy(owned_hbm.at[wid], idx_v)
    _gather(0, 0)
    for c in range(_NCH):
        b = c % 2
        _gather_wait(c, b)
        if c + 1 < _NCH:
            if c >= 1:
                _scatter_wait(c - 1, 1 - b)
            _gather(c + 1, 1 - b)

        def _scale_row(r, carry, _buf=bufs[b]):
            for j in range(_D // _LANES):
                sl = pl.ds(j * _LANES, _LANES)
                _buf[r, sl] = _buf[r, sl] * _factor
            return carry

        lax.fori_loop(0, _CH, _scale_row, 0)
        _scatter(c, b)
    _scatter_wait(_NCH - 2, _NCH % 2)
    _scatter_wait(_NCH - 1, (_NCH - 1) % 2)


_sc_scatter = _mpmd._mpmd_map(
    [(_mesh, _sc_scatter_body)],
    jax.ShapeDtypeStruct((_N_TS, _D), jnp.float32),
    input_output_aliases={0: 0},
    scratch_types=[
        pltpu.VMEM((_NCH, _CH), jnp.int32),
        pltpu.VMEM((_CH, _D), jnp.float32),
        pltpu.VMEM((_CH, _D), jnp.float32),
        pltpu.SemaphoreType.DMA,
        pltpu.SemaphoreType.DMA,
        pltpu.SemaphoreType.DMA,
        pltpu.SemaphoreType.DMA,
    ],
)


def kernel(data):
    if not _apply:
        return data
    copied = _tc_copy(data)
    return _sc_scatter(copied, data, jnp.asarray(_owned))


# speedup vs baseline: 40.8793x; 1.0598x over previous
import functools

import jax
import jax.numpy as jnp
import numpy as np
from jax import lax
from jax.experimental import pallas as pl
from jax.experimental.pallas import tpu as pltpu
from jax.experimental.pallas import tpu_sc as plsc
from jax._src.pallas import mpmd as _mpmd

_P = 1.0
_LB = 0.8
_HB = 1.2
_F = 4096
_N_TS = 65536
_D = 1024

with jax.default_device(jax.local_devices(backend="cpu")[0]):
    _key = jax.random.key(42)
    _k1, _k2, _k3 = jax.random.split(_key, 3)
    _coin = float(jax.random.uniform(_k1, ()))
    _selection = np.asarray(jax.random.choice(_k2, _N_TS, (_F,), replace=False))
    _factor = float((_HB - _LB) * jax.random.uniform(_k3, ()) + _LB)
    _apply = bool(_coin < _P)

_NC = 2
_NS = 16
_NW = _NC * _NS
_RPW = _N_TS // _NW
_CH = 32
_LANES = 16

_NCH = _F // (_NW * _CH)
_owned = _selection.astype(np.int32).reshape(_NW, _NCH, _CH)

_ROWS = 2048


def _tc_copy_body(x_ref, o_ref):
    o_ref[...] = x_ref[...]


def _tc_copy(data):
    return pl.pallas_call(
        _tc_copy_body,
        grid=(_N_TS // _ROWS,),
        in_specs=[pl.BlockSpec((_ROWS, _D), lambda i: (i, 0))],
        out_specs=pl.BlockSpec((_ROWS, _D), lambda i: (i, 0)),
        out_shape=jax.ShapeDtypeStruct((_N_TS, _D), jnp.float32),
    )(data)


_mesh = plsc.VectorSubcoreMesh(core_axis_name="c", subcore_axis_name="s")


def _sc_scatter_body(copied_hbm, data_hbm, owned_hbm, out_hbm, idx_v, buf0,
                     buf1, g0, g1, s0, s1):
    del copied_hbm
    wid = lax.axis_index("s") * _NC + lax.axis_index("c")
    bufs = (buf0, buf1)
    gsems = (g0, g1)
    ssems = (s0, s1)

    def _gather(c, b):
        pltpu.async_copy(data_hbm.at[idx_v.at[c]], bufs[b], gsems[b])

    def _gather_wait(c, b):
        pltpu.make_async_copy(data_hbm.at[idx_v.at[c]], bufs[b],
                              gsems[b]).wait()

    def _scatter(c, b):
        pltpu.async_copy(bufs[b], out_hbm.at[idx_v.at[c]], ssems[b])

    def _scatter_wait(c, b):
        pltpu.make_async_copy(bufs[b], out_hbm.at[idx_v.at[c]],
                              ssems[b]).wait()

    pltpu.sync_copy(owned_hbm.at[wid], idx_v)
    _gather(0, 0)
    for c in range(_NCH):
        b = c % 2
        _gather_wait(c, b)
        if c + 1 < _NCH:
            if c >= 1:
                _scatter_wait(c - 1, 1 - b)
            _gather(c + 1, 1 - b)

        def _scale_row(r, carry, _buf=bufs[b]):
            for j in range(_D // _LANES):
                sl = pl.ds(j * _LANES, _LANES)
                _buf[r, sl] = _buf[r, sl] * _factor
            return carry

        lax.fori_loop(0, _CH, _scale_row, 0)
        _scatter(c, b)
    _scatter_wait(_NCH - 2, _NCH % 2)
    _scatter_wait(_NCH - 1, (_NCH - 1) % 2)


_sc_scatter = _mpmd._mpmd_map(
    [(_mesh, _sc_scatter_body)],
    jax.ShapeDtypeStruct((_N_TS, _D), jnp.float32),
    input_output_aliases={0: 0},
    scratch_types=[
        pltpu.VMEM((_NCH, _CH), jnp.int32),
        pltpu.VMEM((_CH, _D), jnp.float32),
        pltpu.VMEM((_CH, _D), jnp.float32),
        pltpu.SemaphoreType.DMA,
        pltpu.SemaphoreType.DMA,
        pltpu.SemaphoreType.DMA,
        pltpu.SemaphoreType.DMA,
    ],
)


def kernel(data):
    if not _apply:
        return data
    copied = _tc_copy(data)
    return _sc_scatter(copied, data, jnp.asarray(_owned))
